# baseline (speedup 1.0000x reference)
import jax
import jax.numpy as jnp
from jax import lax
from jax.experimental import pallas as pl
from jax.experimental.pallas import tpu as pltpu
from jax.experimental.pallas import tpu_sc as plsc

N = 10000
NP = 10240
E = 160000
D = 256
H = 128
TW = 136
SLOPE = 0.1
CHUNK = 100
NSUPER = 4
SCHUNK = 25
NSUB = 16
ROWS_PER_SUB = NP // NSUB
DIVROWS = 16
EPT = E // 32
WGROUPS = -(-EPT // 16)

_PREP_BLOCK = 1024

_SC_PARAMS = pltpu.CompilerParams(
    needs_layout_passes=False, use_tc_tiling_on_sc=False)


def _prep_body(x_ref, w_ref, b_ref, a_ref, t0_ref, t1_ref, i0_ref, i1_ref,
               s1_ref, s2_ref):
    emb = jnp.dot(x_ref[...], w_ref[...], preferred_element_type=jnp.float32)
    emb = emb + b_ref[...]
    s12 = jnp.dot(emb, a_ref[...], preferred_element_type=jnp.float32)
    s1 = s12[:, 0:1]
    s2 = s12[:, 1:2]
    ssum = s1 + s2
    wl = jnp.exp(jnp.maximum(ssum, ssum * SLOPE))
    col = lax.broadcasted_iota(jnp.int32, (_PREP_BLOCK, TW), 1)
    ones_col = jnp.where(col == H, jnp.float32(1.0), jnp.float32(0.0))
    t0 = jnp.pad(emb[:, :H], ((0, 0), (0, TW - H))) + ones_col
    t1 = jnp.pad(emb[:, H:], ((0, 0), (0, TW - H))) + ones_col
    t0_ref[...] = t0
    t1_ref[...] = t1
    i0_ref[...] = wl * t0
    i1_ref[...] = wl * t1
    s1_ref[...] = s1
    s2_ref[...] = s2


def _prep(x, w, b2, amat):
    nblk = NP // _PREP_BLOCK
    fullspec = pl.BlockSpec((_PREP_BLOCK, TW), lambda i: (i, 0))
    sspec = pl.BlockSpec((_PREP_BLOCK, 1), lambda i: (i, 0))
    return pl.pallas_call(
        _prep_body,
        grid=(nblk,),
        in_specs=[
            pl.BlockSpec((_PREP_BLOCK, D), lambda i: (i, 0)),
            pl.BlockSpec((D, D), lambda i: (0, 0)),
            pl.BlockSpec((1, D), lambda i: (0, 0)),
            pl.BlockSpec((D, 128), lambda i: (0, 0)),
        ],
        out_specs=[fullspec, fullspec, fullspec, fullspec, sspec, sspec],
        out_shape=[
            jax.ShapeDtypeStruct((NP, TW), jnp.float32),
            jax.ShapeDtypeStruct((NP, TW), jnp.float32),
            jax.ShapeDtypeStruct((NP, TW), jnp.float32),
            jax.ShapeDtypeStruct((NP, TW), jnp.float32),
            jax.ShapeDtypeStruct((NP, 1), jnp.float32),
            jax.ShapeDtypeStruct((NP, 1), jnp.float32),
        ],
    )(x, w, b2, amat)


def _wpre_body(s1h, s2h, src1, dst1, wout, s1v, s2v, ib, db, wb):
    c = lax.axis_index("c")
    s = lax.axis_index("s")
    base = (s * 2 + c) * EPT
    pltpu.sync_copy(s1h, s1v)
    pltpu.sync_copy(s2h, s2v)
    pltpu.sync_copy(src1.at[pl.ds(base, EPT)], ib)
    pltpu.sync_copy(dst1.at[pl.ds(base, EPT)], db)

    @pl.loop(0, WGROUPS)
    def _(g):
        off = jnp.minimum(g * 16, EPT - 16)
        sv = ib[pl.ds(off, 16)]
        dv = db[pl.ds(off, 16)]
        lg = plsc.load_gather(s1v, [sv]) + plsc.load_gather(s2v, [dv])
        wb[pl.ds(off, 16)] = jnp.exp(jnp.maximum(lg, lg * SLOPE))

    pltpu.sync_copy(wb, wout.at[pl.ds(base, EPT)])


@jax.jit
def _wpre(s1, s2, src1, dst1):
    mesh = plsc.VectorSubcoreMesh(core_axis_name="c", subcore_axis_name="s")
    return pl.kernel(
        _wpre_body,
        out_type=jax.ShapeDtypeStruct((E,), jnp.float32),
        mesh=mesh,
        compiler_params=_SC_PARAMS,
        scratch_types=[
            pltpu.VMEM((NP,), jnp.float32),
            pltpu.VMEM((NP,), jnp.float32),
            pltpu.VMEM((EPT,), jnp.int32),
            pltpu.VMEM((EPT,), jnp.int32),
            pltpu.VMEM((EPT,), jnp.float32),
        ],
    )(s1, s2, src1, dst1)


def _agg_body(t0, t1, i0, i1, wq, srch, dsth, out,
              acc, isb, idb, wsb, r0b, r1b, dbuf, dbuf1, obuf, obuf1,
              gs0, gs1, ss0, ss1, dsm0, dsm1, osm0, osm1):
    c = lax.axis_index("c")
    s = lax.axis_index("s")
    row0 = s * ROWS_PER_SUB

    @pl.when(c == 0)
    def _():
        pltpu.sync_copy(i0.at[pl.ds(row0, ROWS_PER_SUB)],
                        acc.at[pl.ds(row0, ROWS_PER_SUB)])

    @pl.when(c == 1)
    def _():
        pltpu.sync_copy(i1.at[pl.ds(row0, ROWS_PER_SUB)],
                        acc.at[pl.ds(row0, ROWS_PER_SUB)])

    plsc.subcore_barrier()

    def gather_start(i, buf, sem):
        @pl.when(c == 0)
        def _():
            pltpu.async_copy(t0.at[idb.at[i]], buf, sem)

        @pl.when(c == 1)
        def _():
            pltpu.async_copy(t1.at[idb.at[i]], buf, sem)

    def gather_wait(i, buf, sem):
        pltpu.make_async_copy(t0.at[idb.at[i]], buf, sem).wait()

    def scatter_start(i, buf, sem):
        pltpu.async_copy(buf, acc.at[isb.at[i]], sem, add=True)

    def scatter_wait(i, buf, sem):
        pltpu.make_async_copy(buf, acc.at[isb.at[i]], sem).wait()

    lane = lax.iota(jnp.int32, 16)
    tailmask = lane < 1

    def compute(buf, i):
        iv = jnp.full((16,), i, jnp.int32)

        @pl.loop(0, CHUNK, unroll=4)
        def _(e):
            ev = jnp.full((16,), e, jnp.int32)
            wb = plsc.load_gather(wsb, [iv, ev])
            for k in range(H // 16):
                buf[e, pl.ds(k * 16, 16)] = buf[e, pl.ds(k * 16, 16)] * wb
            plsc.store_scatter(buf, [ev, jnp.full((16,), H, jnp.int32)],
                               wb, mask=tailmask)

    for o in range(NSUPER):
        pltpu.sync_copy(srch.at[s, o], isb)
        pltpu.sync_copy(dsth.at[s, o], idb)
        pltpu.sync_copy(wq.at[s, o], wsb)

        gather_start(0, r0b, gs0)

        @pl.loop(0, (SCHUNK - 1) // 2)
        def _(k):
            i = 2 * k
            gather_start(i + 1, r1b, gs1)
            gather_wait(i, r0b, gs0)
            compute(r0b, i)
            scatter_start(i, r0b, ss0)
            gather_wait(i + 1, r1b, gs1)
            compute(r1b, i + 1)
            scatter_wait(i, r0b, ss0)
            gather_start(i + 2, r0b, gs0)
            scatter_start(i + 1, r1b, ss1)
            scatter_wait(i + 1, r1b, ss1)

        gather_wait(SCHUNK - 1, r0b, gs0)
        compute(r0b, SCHUNK - 1)
        scatter_start(SCHUNK - 1, r0b, ss0)
        scatter_wait(SCHUNK - 1, r0b, ss0)

    plsc.subcore_barrier()

    hv = jnp.full((16,), H, jnp.int32)

    def din_start(a, d, sem):
        pltpu.async_copy(acc.at[pl.ds(row0 + a * DIVROWS, DIVROWS)], d, sem)

    def din_wait(a, d, sem):
        pltpu.make_async_copy(
            acc.at[pl.ds(row0 + a * DIVROWS, DIVROWS)], d, sem).wait()

    def dcompute(d, ob):
        @pl.loop(0, DIVROWS)
        def _(r):
            rv = jnp.full((16,), r, jnp.int32)
            inv = jnp.float32(1.0) / plsc.load_gather(d, [rv, hv])
            for k in range(H // 16):
                ob[r, pl.ds(k * 16, 16)] = d[r, pl.ds(k * 16, 16)] * inv

    def dout_start(a, ob, sem):
        @pl.when(row0 + a * DIVROWS < N)
        def _():
            pltpu.async_copy(
                ob, out.at[pl.ds(row0 + a * DIVROWS, DIVROWS),
                           pl.ds(c * H, H)], sem)

    def dout_wait(a, ob, sem):
        @pl.when(jnp.logical_and(a >= 0, row0 + a * DIVROWS < N))
        def _():
            pltpu.make_async_copy(
                ob, out.at[pl.ds(row0 + a * DIVROWS, DIVROWS),
                           pl.ds(c * H, H)], sem).wait()

    nslab = ROWS_PER_SUB // DIVROWS
    din_start(0, dbuf, dsm0)

    @pl.loop(0, nslab // 2)
    def _(k):
        a = 2 * k
        b = a + 1
        din_start(b, dbuf1, dsm1)
        din_wait(a, dbuf, dsm0)
        dout_wait(a - 2, obuf, osm0)
        dcompute(dbuf, obuf)
        dout_start(a, obuf, osm0)

        @pl.when(k < nslab // 2 - 1)
        def _():
            din_start(a + 2, dbuf, dsm0)

        din_wait(b, dbuf1, dsm1)
        dout_wait(b - 2, obuf1, osm1)
        dcompute(dbuf1, obuf1)
        dout_start(b, obuf1, osm1)

    dout_wait(nslab - 2, obuf, osm0)
    dout_wait(nslab - 1, obuf1, osm1)


@jax.jit
def _agg(t0, t1, i0, i1, wq, src, dst):
    mesh = plsc.VectorSubcoreMesh(core_axis_name="c", subcore_axis_name="s")
    return pl.kernel(
        _agg_body,
        out_type=jax.ShapeDtypeStruct((N, D), jnp.float32),
        mesh=mesh,
        compiler_params=_SC_PARAMS,
        scratch_types=[
            pltpu.VMEM_SHARED((NP, TW), jnp.float32),
            pltpu.VMEM((SCHUNK, CHUNK), jnp.int32),
            pltpu.VMEM((SCHUNK, CHUNK), jnp.int32),
            pltpu.VMEM((SCHUNK, CHUNK), jnp.float32),
            pltpu.VMEM((CHUNK, TW), jnp.float32),
            pltpu.VMEM((CHUNK, TW), jnp.float32),
            pltpu.VMEM((DIVROWS, TW), jnp.float32),
            pltpu.VMEM((DIVROWS, TW), jnp.float32),
            pltpu.VMEM((DIVROWS, H), jnp.float32),
            pltpu.VMEM((DIVROWS, H), jnp.float32),
        ] + [pltpu.SemaphoreType.DMA] * 8,
    )(t0, t1, i0, i1, wq, src, dst)


def kernel(x, edge_index, W, b, a):
    amat = jnp.zeros((D, 128), jnp.float32)
    amat = amat.at[:, 0].set(a[:D, 0]).at[:, 1].set(a[D:, 0])
    b2 = b.reshape(1, D)
    t0, t1, i0, i1, s1, s2 = _prep(x, W, b2, amat)
    w1 = _wpre(s1.reshape(NP), s2.reshape(NP), edge_index[0], edge_index[1])
    wq = w1.reshape(NSUB, NSUPER, SCHUNK, CHUNK)
    src = edge_index[0].reshape(NSUB, NSUPER, SCHUNK, CHUNK)
    dst = edge_index[1].reshape(NSUB, NSUPER, SCHUNK, CHUNK)
    return _agg(t0, t1, i0, i1, wq, src, dst)


# speedup vs baseline: 1.0087x; 1.0087x over previous
import jax
import jax.numpy as jnp
from jax import lax
from jax.experimental import pallas as pl
from jax.experimental.pallas import tpu as pltpu
from jax.experimental.pallas import tpu_sc as plsc

N = 10000
NP = 10240
E = 160000
D = 256
H = 128
TW = 144
SLOPE = 0.1
CHUNK = 80
NSUPER = 5
SCHUNK = 25
NSUB = 16
ROWS_PER_SUB = NP // NSUB
DIVROWS = 16
EPT = E // 32
WGROUPS = -(-EPT // 16)

_PREP_BLOCK = 1024

_SC_PARAMS = pltpu.CompilerParams(
    needs_layout_passes=False, use_tc_tiling_on_sc=False)


def _prep_body(x_ref, w_ref, b_ref, a_ref, t0_ref, t1_ref, i0_ref, i1_ref,
               s1_ref, s2_ref):
    emb = jnp.dot(x_ref[...], w_ref[...], preferred_element_type=jnp.float32)
    emb = emb + b_ref[...]
    s12 = jnp.dot(emb, a_ref[...], preferred_element_type=jnp.float32)
    s1 = s12[:, 0:1]
    s2 = s12[:, 1:2]
    ssum = s1 + s2
    wl = jnp.exp(jnp.maximum(ssum, ssum * SLOPE))
    col = lax.broadcasted_iota(jnp.int32, (_PREP_BLOCK, TW), 1)
    ones_col = jnp.where(col == H, jnp.float32(1.0), jnp.float32(0.0))
    t0 = jnp.pad(emb[:, :H], ((0, 0), (0, TW - H))) + ones_col
    t1 = jnp.pad(emb[:, H:], ((0, 0), (0, TW - H))) + ones_col
    t0_ref[...] = t0
    t1_ref[...] = t1
    i0_ref[...] = wl * t0
    i1_ref[...] = wl * t1
    s1_ref[...] = s1
    s2_ref[...] = s2


def _prep(x, w, b2, amat):
    nblk = NP // _PREP_BLOCK
    fullspec = pl.BlockSpec((_PREP_BLOCK, TW), lambda i: (i, 0))
    sspec = pl.BlockSpec((_PREP_BLOCK, 1), lambda i: (i, 0))
    return pl.pallas_call(
        _prep_body,
        grid=(nblk,),
        in_specs=[
            pl.BlockSpec((_PREP_BLOCK, D), lambda i: (i, 0)),
            pl.BlockSpec((D, D), lambda i: (0, 0)),
            pl.BlockSpec((1, D), lambda i: (0, 0)),
            pl.BlockSpec((D, 128), lambda i: (0, 0)),
        ],
        out_specs=[fullspec, fullspec, fullspec, fullspec, sspec, sspec],
        out_shape=[
            jax.ShapeDtypeStruct((NP, TW), jnp.float32),
            jax.ShapeDtypeStruct((NP, TW), jnp.float32),
            jax.ShapeDtypeStruct((NP, TW), jnp.float32),
            jax.ShapeDtypeStruct((NP, TW), jnp.float32),
            jax.ShapeDtypeStruct((NP, 1), jnp.float32),
            jax.ShapeDtypeStruct((NP, 1), jnp.float32),
        ],
    )(x, w, b2, amat)


def _wpre_body(s1h, s2h, src1, dst1, wout, s1v, s2v, ib, db, wb):
    c = lax.axis_index("c")
    s = lax.axis_index("s")
    base = (s * 2 + c) * EPT
    pltpu.sync_copy(s1h, s1v)
    pltpu.sync_copy(s2h, s2v)
    pltpu.sync_copy(src1.at[pl.ds(base, EPT)], ib)
    pltpu.sync_copy(dst1.at[pl.ds(base, EPT)], db)

    @pl.loop(0, WGROUPS)
    def _(g):
        off = jnp.minimum(g * 16, EPT - 16)
        sv = ib[pl.ds(off, 16)]
        dv = db[pl.ds(off, 16)]
        lg = plsc.load_gather(s1v, [sv]) + plsc.load_gather(s2v, [dv])
        wb[pl.ds(off, 16)] = jnp.exp(jnp.maximum(lg, lg * SLOPE))

    pltpu.sync_copy(wb, wout.at[pl.ds(base, EPT)])


@jax.jit
def _wpre(s1, s2, src1, dst1):
    mesh = plsc.VectorSubcoreMesh(core_axis_name="c", subcore_axis_name="s")
    return pl.kernel(
        _wpre_body,
        out_type=jax.ShapeDtypeStruct((E,), jnp.float32),
        mesh=mesh,
        compiler_params=_SC_PARAMS,
        scratch_types=[
            pltpu.VMEM((NP,), jnp.float32),
            pltpu.VMEM((NP,), jnp.float32),
            pltpu.VMEM((EPT,), jnp.int32),
            pltpu.VMEM((EPT,), jnp.int32),
            pltpu.VMEM((EPT,), jnp.float32),
        ],
    )(s1, s2, src1, dst1)


def _agg_body(t0, t1, i0, i1, wq, srch, dsth, out,
              acc, isb, idb, wsb, r0b, r1b, dbuf, dbuf1, obuf, obuf1,
              gs0, gs1, ss0, ss1, dsm0, dsm1, osm0, osm1):
    c = lax.axis_index("c")
    s = lax.axis_index("s")
    row0 = s * ROWS_PER_SUB

    @pl.when(c == 0)
    def _():
        pltpu.sync_copy(i0.at[pl.ds(row0, ROWS_PER_SUB)],
                        acc.at[pl.ds(row0, ROWS_PER_SUB)])

    @pl.when(c == 1)
    def _():
        pltpu.sync_copy(i1.at[pl.ds(row0, ROWS_PER_SUB)],
                        acc.at[pl.ds(row0, ROWS_PER_SUB)])

    plsc.subcore_barrier()

    def gather_start(i, buf, sem):
        @pl.when(c == 0)
        def _():
            pltpu.async_copy(t0.at[idb.at[i]], buf, sem)

        @pl.when(c == 1)
        def _():
            pltpu.async_copy(t1.at[idb.at[i]], buf, sem)

    def gather_wait(i, buf, sem):
        pltpu.make_async_copy(t0.at[idb.at[i]], buf, sem).wait()

    def scatter_start(i, buf, sem):
        pltpu.async_copy(buf, acc.at[isb.at[i]], sem, add=True)

    def scatter_wait(i, buf, sem):
        pltpu.make_async_copy(buf, acc.at[isb.at[i]], sem).wait()

    lane = lax.iota(jnp.int32, 16)
    tailmask = lane < 1

    def compute(buf, i):
        iv = jnp.full((16,), i, jnp.int32)

        @pl.loop(0, CHUNK, unroll=4)
        def _(e):
            ev = jnp.full((16,), e, jnp.int32)
            wb = plsc.load_gather(wsb, [iv, ev])
            for k in range(H // 16):
                buf[e, pl.ds(k * 16, 16)] = buf[e, pl.ds(k * 16, 16)] * wb
            plsc.store_scatter(buf, [ev, jnp.full((16,), H, jnp.int32)],
                               wb, mask=tailmask)

    for o in range(NSUPER):
        pltpu.sync_copy(srch.at[s, o], isb)
        pltpu.sync_copy(dsth.at[s, o], idb)
        pltpu.sync_copy(wq.at[s, o], wsb)

        gather_start(0, r0b, gs0)

        @pl.loop(0, (SCHUNK - 1) // 2)
        def _(k):
            i = 2 * k
            gather_start(i + 1, r1b, gs1)
            gather_wait(i, r0b, gs0)
            compute(r0b, i)
            scatter_start(i, r0b, ss0)
            gather_wait(i + 1, r1b, gs1)
            compute(r1b, i + 1)
            scatter_wait(i, r0b, ss0)
            gather_start(i + 2, r0b, gs0)
            scatter_start(i + 1, r1b, ss1)
            scatter_wait(i + 1, r1b, ss1)

        gather_wait(SCHUNK - 1, r0b, gs0)
        compute(r0b, SCHUNK - 1)
        scatter_start(SCHUNK - 1, r0b, ss0)
        scatter_wait(SCHUNK - 1, r0b, ss0)

    plsc.subcore_barrier()

    hv = jnp.full((16,), H, jnp.int32)

    def din_start(a, d, sem):
        pltpu.async_copy(acc.at[pl.ds(row0 + a * DIVROWS, DIVROWS)], d, sem)

    def din_wait(a, d, sem):
        pltpu.make_async_copy(
            acc.at[pl.ds(row0 + a * DIVROWS, DIVROWS)], d, sem).wait()

    def dcompute(d, ob):
        @pl.loop(0, DIVROWS)
        def _(r):
            rv = jnp.full((16,), r, jnp.int32)
            inv = jnp.float32(1.0) / plsc.load_gather(d, [rv, hv])
            for k in range(H // 16):
                ob[r, pl.ds(k * 16, 16)] = d[r, pl.ds(k * 16, 16)] * inv

    def dout_start(a, ob, sem):
        @pl.when(row0 + a * DIVROWS < N)
        def _():
            pltpu.async_copy(
                ob, out.at[pl.ds(row0 + a * DIVROWS, DIVROWS),
                           pl.ds(c * H, H)], sem)

    def dout_wait(a, ob, sem):
        @pl.when(jnp.logical_and(a >= 0, row0 + a * DIVROWS < N))
        def _():
            pltpu.make_async_copy(
                ob, out.at[pl.ds(row0 + a * DIVROWS, DIVROWS),
                           pl.ds(c * H, H)], sem).wait()

    nslab = ROWS_PER_SUB // DIVROWS
    din_start(0, dbuf, dsm0)

    @pl.loop(0, nslab // 2)
    def _(k):
        a = 2 * k
        b = a + 1
        din_start(b, dbuf1, dsm1)
        din_wait(a, dbuf, dsm0)
        dout_wait(a - 2, obuf, osm0)
        dcompute(dbuf, obuf)
        dout_start(a, obuf, osm0)

        @pl.when(k < nslab // 2 - 1)
        def _():
            din_start(a + 2, dbuf, dsm0)

        din_wait(b, dbuf1, dsm1)
        dout_wait(b - 2, obuf1, osm1)
        dcompute(dbuf1, obuf1)
        dout_start(b, obuf1, osm1)

    dout_wait(nslab - 2, obuf, osm0)
    dout_wait(nslab - 1, obuf1, osm1)


@jax.jit
def _agg(t0, t1, i0, i1, wq, src, dst):
    mesh = plsc.VectorSubcoreMesh(core_axis_name="c", subcore_axis_name="s")
    return pl.kernel(
        _agg_body,
        out_type=jax.ShapeDtypeStruct((N, D), jnp.float32),
        mesh=mesh,
        compiler_params=_SC_PARAMS,
        scratch_types=[
            pltpu.VMEM_SHARED((NP, TW), jnp.float32),
            pltpu.VMEM((SCHUNK, CHUNK), jnp.int32),
            pltpu.VMEM((SCHUNK, CHUNK), jnp.int32),
            pltpu.VMEM((SCHUNK, CHUNK), jnp.float32),
            pltpu.VMEM((CHUNK, TW), jnp.float32),
            pltpu.VMEM((CHUNK, TW), jnp.float32),
            pltpu.VMEM((DIVROWS, TW), jnp.float32),
            pltpu.VMEM((DIVROWS, TW), jnp.float32),
            pltpu.VMEM((DIVROWS, H), jnp.float32),
            pltpu.VMEM((DIVROWS, H), jnp.float32),
        ] + [pltpu.SemaphoreType.DMA] * 8,
    )(t0, t1, i0, i1, wq, src, dst)


def kernel(x, edge_index, W, b, a):
    amat = jnp.zeros((D, 128), jnp.float32)
    amat = amat.at[:, 0].set(a[:D, 0]).at[:, 1].set(a[D:, 0])
    b2 = b.reshape(1, D)
    t0, t1, i0, i1, s1, s2 = _prep(x, W, b2, amat)
    w1 = _wpre(s1.reshape(NP), s2.reshape(NP), edge_index[0], edge_index[1])
    wq = w1.reshape(NSUB, NSUPER, SCHUNK, CHUNK)
    src = edge_index[0].reshape(NSUB, NSUPER, SCHUNK, CHUNK)
    dst = edge_index[1].reshape(NSUB, NSUPER, SCHUNK, CHUNK)
    return _agg(t0, t1, i0, i1, wq, src, dst)


# speedup vs baseline: 1.6969x; 1.6822x over previous
import jax
import jax.numpy as jnp
from jax import lax
from jax.experimental import pallas as pl
from jax.experimental.pallas import tpu as pltpu
from jax.experimental.pallas import tpu_sc as plsc

N = 10000
NP = 10240
E = 160000
D = 256
H = 128
TW = 144
SLOPE = 0.1
CHUNK = 80
NSUPER = 5
SCHUNK = 25
NSUB = 16
ROWS_PER_SUB = NP // NSUB
DIVROWS = 16
EPT = E // 32
WGROUPS = -(-EPT // 16)

_PREP_BLOCK = 1024

_SC_PARAMS = pltpu.CompilerParams(
    needs_layout_passes=False, use_tc_tiling_on_sc=False)


def _prep_body(x_ref, w_ref, b_ref, a_ref, t0_ref, t1_ref, i0_ref, i1_ref,
               s1_ref, s2_ref):
    emb = jnp.dot(x_ref[...], w_ref[...], preferred_element_type=jnp.float32)
    emb = emb + b_ref[...]
    s12 = jnp.dot(emb, a_ref[...], preferred_element_type=jnp.float32)
    s1 = s12[:, 0:1]
    s2 = s12[:, 1:2]
    ssum = s1 + s2
    wl = jnp.exp(jnp.maximum(ssum, ssum * SLOPE))
    col = lax.broadcasted_iota(jnp.int32, (_PREP_BLOCK, TW), 1)
    ones_col = jnp.where(col == H, jnp.float32(1.0), jnp.float32(0.0))
    t0 = jnp.pad(emb[:, :H], ((0, 0), (0, TW - H))) + ones_col
    t1 = jnp.pad(emb[:, H:], ((0, 0), (0, TW - H))) + ones_col
    t0_ref[...] = t0
    t1_ref[...] = t1
    i0_ref[...] = wl * t0
    i1_ref[...] = wl * t1
    s1_ref[...] = s1
    s2_ref[...] = s2


def _prep(x, w, b2, amat):
    nblk = NP // _PREP_BLOCK
    fullspec = pl.BlockSpec((_PREP_BLOCK, TW), lambda i: (i, 0))
    sspec = pl.BlockSpec((_PREP_BLOCK, 1), lambda i: (i, 0))
    return pl.pallas_call(
        _prep_body,
        grid=(nblk,),
        in_specs=[
            pl.BlockSpec((_PREP_BLOCK, D), lambda i: (i, 0)),
            pl.BlockSpec((D, D), lambda i: (0, 0)),
            pl.BlockSpec((1, D), lambda i: (0, 0)),
            pl.BlockSpec((D, 128), lambda i: (0, 0)),
        ],
        out_specs=[fullspec, fullspec, fullspec, fullspec, sspec, sspec],
        out_shape=[
            jax.ShapeDtypeStruct((NP, TW), jnp.float32),
            jax.ShapeDtypeStruct((NP, TW), jnp.float32),
            jax.ShapeDtypeStruct((NP, TW), jnp.float32),
            jax.ShapeDtypeStruct((NP, TW), jnp.float32),
            jax.ShapeDtypeStruct((NP, 1), jnp.float32),
            jax.ShapeDtypeStruct((NP, 1), jnp.float32),
        ],
    )(x, w, b2, amat)


def _wpre_body(s1h, s2h, src1, dst1, wout, s1v, s2v, ib, db, wb):
    c = lax.axis_index("c")
    s = lax.axis_index("s")
    base = (s * 2 + c) * EPT
    pltpu.sync_copy(s1h, s1v)
    pltpu.sync_copy(s2h, s2v)
    pltpu.sync_copy(src1.at[pl.ds(base, EPT)], ib)
    pltpu.sync_copy(dst1.at[pl.ds(base, EPT)], db)

    @pl.loop(0, WGROUPS)
    def _(g):
        off = jnp.minimum(g * 16, EPT - 16)
        sv = ib[pl.ds(off, 16)]
        dv = db[pl.ds(off, 16)]
        lg = plsc.load_gather(s1v, [sv]) + plsc.load_gather(s2v, [dv])
        wb[pl.ds(off, 16)] = jnp.exp(jnp.maximum(lg, lg * SLOPE))

    pltpu.sync_copy(wb, wout.at[pl.ds(base, EPT)])


@jax.jit
def _wpre(s1, s2, src1, dst1):
    mesh = plsc.VectorSubcoreMesh(core_axis_name="c", subcore_axis_name="s")
    return pl.kernel(
        _wpre_body,
        out_type=jax.ShapeDtypeStruct((E,), jnp.float32),
        mesh=mesh,
        compiler_params=_SC_PARAMS,
        scratch_types=[
            pltpu.VMEM((NP,), jnp.float32),
            pltpu.VMEM((NP,), jnp.float32),
            pltpu.VMEM((EPT,), jnp.int32),
            pltpu.VMEM((EPT,), jnp.int32),
            pltpu.VMEM((EPT,), jnp.float32),
        ],
    )(s1, s2, src1, dst1)


def _agg_body(t0, t1, i0, i1, wq, srch, dsth, out,
              acc, isb, idb, wsb, r0b, r1b, dbuf, dbuf1, obuf, obuf1,
              gs0, gs1, ss0, ss1, dsm0, dsm1, osm0, osm1):
    c = lax.axis_index("c")
    s = lax.axis_index("s")
    row0 = s * ROWS_PER_SUB

    @pl.when(c == 0)
    def _():
        pltpu.sync_copy(i0.at[pl.ds(row0, ROWS_PER_SUB)],
                        acc.at[pl.ds(row0, ROWS_PER_SUB)])

    @pl.when(c == 1)
    def _():
        pltpu.sync_copy(i1.at[pl.ds(row0, ROWS_PER_SUB)],
                        acc.at[pl.ds(row0, ROWS_PER_SUB)])

    plsc.subcore_barrier()

    def gather_start(i, buf, sem):
        @pl.when(c == 0)
        def _():
            pltpu.async_copy(t0.at[idb.at[i]], buf, sem)

        @pl.when(c == 1)
        def _():
            pltpu.async_copy(t1.at[idb.at[i]], buf, sem)

    def gather_wait(i, buf, sem):
        pltpu.make_async_copy(t0.at[idb.at[i]], buf, sem).wait()

    def scatter_start(i, buf, sem):
        pltpu.async_copy(buf, acc.at[isb.at[i]], sem, add=True)

    def scatter_wait(i, buf, sem):
        pltpu.make_async_copy(buf, acc.at[isb.at[i]], sem).wait()

    def compute(buf, i):
        iv = jnp.full((16,), i, jnp.int32)

        @pl.loop(0, CHUNK, unroll=4)
        def _(e):
            wb = plsc.load_gather(wsb, [iv, jnp.full((16,), e, jnp.int32)])
            for k in range(TW // 16):
                buf[e, pl.ds(k * 16, 16)] = buf[e, pl.ds(k * 16, 16)] * wb

    for o in range(NSUPER):
        pltpu.sync_copy(srch.at[s, o], isb)
        pltpu.sync_copy(dsth.at[s, o], idb)
        pltpu.sync_copy(wq.at[s, o], wsb)

        gather_start(0, r0b, gs0)

        @pl.loop(0, (SCHUNK - 1) // 2)
        def _(k):
            i = 2 * k
            gather_start(i + 1, r1b, gs1)
            gather_wait(i, r0b, gs0)
            compute(r0b, i)
            scatter_start(i, r0b, ss0)
            gather_wait(i + 1, r1b, gs1)
            compute(r1b, i + 1)
            scatter_wait(i, r0b, ss0)
            gather_start(i + 2, r0b, gs0)
            scatter_start(i + 1, r1b, ss1)
            scatter_wait(i + 1, r1b, ss1)

        gather_wait(SCHUNK - 1, r0b, gs0)
        compute(r0b, SCHUNK - 1)
        scatter_start(SCHUNK - 1, r0b, ss0)
        scatter_wait(SCHUNK - 1, r0b, ss0)

    plsc.subcore_barrier()

    hv = jnp.full((16,), H, jnp.int32)

    def din_start(a, d, sem):
        pltpu.async_copy(acc.at[pl.ds(row0 + a * DIVROWS, DIVROWS)], d, sem)

    def din_wait(a, d, sem):
        pltpu.make_async_copy(
            acc.at[pl.ds(row0 + a * DIVROWS, DIVROWS)], d, sem).wait()

    def dcompute(d, ob):
        @pl.loop(0, DIVROWS)
        def _(r):
            rv = jnp.full((16,), r, jnp.int32)
            inv = jnp.float32(1.0) / plsc.load_gather(d, [rv, hv])
            for k in range(H // 16):
                ob[r, pl.ds(k * 16, 16)] = d[r, pl.ds(k * 16, 16)] * inv

    def dout_start(a, ob, sem):
        @pl.when(row0 + a * DIVROWS < N)
        def _():
            pltpu.async_copy(
                ob, out.at[pl.ds(row0 + a * DIVROWS, DIVROWS),
                           pl.ds(c * H, H)], sem)

    def dout_wait(a, ob, sem):
        @pl.when(jnp.logical_and(a >= 0, row0 + a * DIVROWS < N))
        def _():
            pltpu.make_async_copy(
                ob, out.at[pl.ds(row0 + a * DIVROWS, DIVROWS),
                           pl.ds(c * H, H)], sem).wait()

    nslab = ROWS_PER_SUB // DIVROWS
    din_start(0, dbuf, dsm0)

    @pl.loop(0, nslab // 2)
    def _(k):
        a = 2 * k
        b = a + 1
        din_start(b, dbuf1, dsm1)
        din_wait(a, dbuf, dsm0)
        dout_wait(a - 2, obuf, osm0)
        dcompute(dbuf, obuf)
        dout_start(a, obuf, osm0)

        @pl.when(k < nslab // 2 - 1)
        def _():
            din_start(a + 2, dbuf, dsm0)

        din_wait(b, dbuf1, dsm1)
        dout_wait(b - 2, obuf1, osm1)
        dcompute(dbuf1, obuf1)
        dout_start(b, obuf1, osm1)

    dout_wait(nslab - 2, obuf, osm0)
    dout_wait(nslab - 1, obuf1, osm1)


@jax.jit
def _agg(t0, t1, i0, i1, wq, src, dst):
    mesh = plsc.VectorSubcoreMesh(core_axis_name="c", subcore_axis_name="s")
    return pl.kernel(
        _agg_body,
        out_type=jax.ShapeDtypeStruct((N, D), jnp.float32),
        mesh=mesh,
        compiler_params=_SC_PARAMS,
        scratch_types=[
            pltpu.VMEM_SHARED((NP, TW), jnp.float32),
            pltpu.VMEM((SCHUNK, CHUNK), jnp.int32),
            pltpu.VMEM((SCHUNK, CHUNK), jnp.int32),
            pltpu.VMEM((SCHUNK, CHUNK), jnp.float32),
            pltpu.VMEM((CHUNK, TW), jnp.float32),
            pltpu.VMEM((CHUNK, TW), jnp.float32),
            pltpu.VMEM((DIVROWS, TW), jnp.float32),
            pltpu.VMEM((DIVROWS, TW), jnp.float32),
            pltpu.VMEM((DIVROWS, H), jnp.float32),
            pltpu.VMEM((DIVROWS, H), jnp.float32),
        ] + [pltpu.SemaphoreType.DMA] * 8,
    )(t0, t1, i0, i1, wq, src, dst)


def kernel(x, edge_index, W, b, a):
    amat = jnp.zeros((D, 128), jnp.float32)
    amat = amat.at[:, 0].set(a[:D, 0]).at[:, 1].set(a[D:, 0])
    b2 = b.reshape(1, D)
    t0, t1, i0, i1, s1, s2 = _prep(x, W, b2, amat)
    w1 = _wpre(s1.reshape(NP), s2.reshape(NP), edge_index[0], edge_index[1])
    wq = w1.reshape(NSUB, NSUPER, SCHUNK, CHUNK)
    src = edge_index[0].reshape(NSUB, NSUPER, SCHUNK, CHUNK)
    dst = edge_index[1].reshape(NSUB, NSUPER, SCHUNK, CHUNK)
    return _agg(t0, t1, i0, i1, wq, src, dst)
